# Initial kernel scaffold; baseline (speedup 1.0000x reference)
import jax
import jax.numpy as jnp
from jax.experimental import pallas as pl


def kernel(xyz, features, fc1_w, fc1_b, fc2_w, fc2_b, d1_w, d1_b, d2_w, d2_b, g1_w, g1_b, g2_w, g2_b, wq, wk, wv):
    raise NotImplementedError("write your pallas kernel here")


# speedup vs baseline: 11.3442x; 11.3442x over previous
import functools

import jax
import jax.numpy as jnp
from jax import lax
from jax.experimental import pallas as pl
from jax.experimental.pallas import tpu as pltpu
from jax.experimental.pallas import tpu_sc as plsc

B, N, K, DP, DM = 2, 2048, 16, 128, 256
BN = B * N

_MR = 512


def _qkv_body(feat_ref, fc1w_ref, fc1b_ref, wq_ref, wk_ref, wv_ref,
              q_ref, xk_ref, xv_ref):
    x = jnp.dot(feat_ref[...], fc1w_ref[...],
                preferred_element_type=jnp.float32) + fc1b_ref[...]
    q_ref[...] = jnp.dot(x, wq_ref[...], preferred_element_type=jnp.float32)
    xk_ref[...] = jnp.dot(x, wk_ref[...], preferred_element_type=jnp.float32)
    xv_ref[...] = jnp.dot(x, wv_ref[...], preferred_element_type=jnp.float32)


def _qkv(feat, fc1_w, fc1_b, wq, wk, wv):
    grid = (BN // _MR,)
    full = lambda i: (0, 0)
    return pl.pallas_call(
        _qkv_body,
        grid=grid,
        in_specs=[
            pl.BlockSpec((_MR, DP), lambda i: (i, 0)),
            pl.BlockSpec((DP, DM), full),
            pl.BlockSpec((1, DM), full),
            pl.BlockSpec((DM, DM), full),
            pl.BlockSpec((DM, DM), full),
            pl.BlockSpec((DM, DM), full),
        ],
        out_specs=[pl.BlockSpec((_MR, DM), lambda i: (i, 0))] * 3,
        out_shape=[jax.ShapeDtypeStruct((BN, DM), jnp.float32)] * 3,
    )(feat, fc1_w, fc1_b, wq, wk, wv)


_MQ = 256


def _knn_body(xyz_ref, xyzT_ref, idx_ref):
    b = pl.program_id(0)
    xq = xyz_ref[0]
    xa = xyzT_ref[0]
    d0 = xq[:, 0:1] - xa[0:1, :]
    d1 = xq[:, 1:2] - xa[1:2, :]
    d2 = xq[:, 2:3] - xa[2:3, :]
    D = d0 * d0 + d1 * d1 + d2 * d2
    iota = lax.broadcasted_iota(jnp.int32, (_MQ, N), 1)
    cols = []
    for _ in range(K):
        mval = jnp.min(D, axis=1, keepdims=True)
        idxk = jnp.min(jnp.where(D == mval, iota, N), axis=1, keepdims=True)
        cols.append(idxk)
        D = jnp.where(iota == idxk, jnp.float32(jnp.inf), D)
    idx_ref[...] = jnp.concatenate(cols, axis=1) + b * N


def _knn(xyz, xyzT):
    nb = N // _MQ
    return pl.pallas_call(
        _knn_body,
        grid=(B, nb),
        in_specs=[
            pl.BlockSpec((1, _MQ, 3), lambda b, m: (b, m, 0)),
            pl.BlockSpec((1, 3, N), lambda b, m: (b, 0, 0)),
        ],
        out_specs=pl.BlockSpec((_MQ, K), lambda b, m: (b * nb + m, 0)),
        out_shape=jax.ShapeDtypeStruct((BN, K), jnp.int32),
    )(xyz, xyzT)


_NC, _NS = 2, 16
_NW = _NC * _NS
_TOT = BN * K
_PW = _TOT // _NW
_CH = 128
_NCH = _PW // _CH


def _gather_body(idx_hbm, xk_hbm, xv_hbm, xyz_hbm, okf, ov, oxy,
                 idx_v, bk, bv, bx, sk, sv, sx):
    wid = lax.axis_index("s") * _NC + lax.axis_index("c")
    base = wid * _PW
    pltpu.sync_copy(idx_hbm.at[pl.ds(base, _PW)], idx_v)

    def chunk(c, carry):
        off = pl.multiple_of(c * _CH, _CH)
        ii = idx_v.at[pl.ds(off, _CH)]
        ck = pltpu.async_copy(xk_hbm.at[ii], bk, sk)
        cv = pltpu.async_copy(xv_hbm.at[ii], bv, sv)
        cx = pltpu.async_copy(xyz_hbm.at[ii], bx, sx)
        ck.wait()
        cv.wait()
        cx.wait()
        gbase = pl.multiple_of(base + off, _CH)
        pltpu.sync_copy(bk, okf.at[pl.ds(gbase, _CH)])
        pltpu.sync_copy(bv, ov.at[pl.ds(gbase, _CH)])
        pltpu.sync_copy(bx, oxy.at[pl.ds(gbase, _CH)])
        return carry

    lax.fori_loop(0, _NCH, chunk, 0)


def _sc_gather(idx_flat, xk, xv, xyzp):
    mesh = plsc.VectorSubcoreMesh(core_axis_name="c", subcore_axis_name="s")
    fn = pl.kernel(
        _gather_body,
        out_type=(
            jax.ShapeDtypeStruct((_TOT, DM), jnp.float32),
            jax.ShapeDtypeStruct((_TOT, DM), jnp.float32),
            jax.ShapeDtypeStruct((_TOT, 128), jnp.float32),
        ),
        mesh=mesh,
        scratch_types=[
            pltpu.VMEM((_PW,), jnp.int32),
            pltpu.VMEM((_CH, DM), jnp.float32),
            pltpu.VMEM((_CH, DM), jnp.float32),
            pltpu.VMEM((_CH, 128), jnp.float32),
            pltpu.SemaphoreType.DMA,
            pltpu.SemaphoreType.DMA,
            pltpu.SemaphoreType.DMA,
        ],
    )
    return fn(idx_flat, xk, xv, xyzp)


_MB = 128
_MK = _MB * K


def _attn_body(q_ref, feat_ref, xyzq_ref, kf_ref, v_ref, xyzg_ref,
               d1_ref, d1b_ref, d2_ref, d2b_ref, g1_ref, g1b_ref,
               g2_ref, g2b_ref, fc2_ref, fc2b_ref, res_ref, attn_ref):
    xq = xyzq_ref[...]
    xg = xyzg_ref[...]
    pos1 = d1b_ref[...]
    for d in range(3):
        qd = jnp.reshape(
            jnp.broadcast_to(xq[:, d:d + 1].reshape(_MB, 1, 1), (_MB, K, 1)),
            (_MK, 1))
        reld = qd - xg[:, d:d + 1]
        pos1 = pos1 + reld * d1_ref[d:d + 1, :]
    h1 = jnp.maximum(pos1, 0.0)
    pos = jnp.dot(h1, d2_ref[...],
                  preferred_element_type=jnp.float32) + d2b_ref[...]

    qb = q_ref[...]
    qrep = jnp.reshape(
        jnp.broadcast_to(qb.reshape(_MB, 1, DM), (_MB, K, DM)), (_MK, DM))
    a_in = qrep - kf_ref[...] + pos
    h2 = jnp.maximum(
        jnp.dot(a_in, g1_ref[...], preferred_element_type=jnp.float32)
        + g1b_ref[...], 0.0)
    t = jnp.dot(h2, g2_ref[...],
                preferred_element_type=jnp.float32) + g2b_ref[...]
    s3 = (t * (1.0 / 16.0)).reshape(_MB, K, DM)
    m = jnp.max(s3, axis=1, keepdims=True)
    e = jnp.exp(s3 - m)
    attn3 = e / jnp.sum(e, axis=1, keepdims=True)
    attn_ref[...] = attn3
    vp = (v_ref[...] + pos).reshape(_MB, K, DM)
    r = jnp.sum(attn3 * vp, axis=1)
    res_ref[...] = (jnp.dot(r, fc2_ref[...], preferred_element_type=jnp.float32)
                    + fc2b_ref[...] + feat_ref[...])


def _attn(q, feat, xyzp, kf, vg, xg, d1_w, d1_b, d2_w, d2_b,
          g1_w, g1_b, g2_w, g2_b, fc2_w, fc2_b):
    grid = (BN // _MB,)
    full = lambda i: (0, 0)
    return pl.pallas_call(
        _attn_body,
        grid=grid,
        in_specs=[
            pl.BlockSpec((_MB, DM), lambda i: (i, 0)),
            pl.BlockSpec((_MB, DP), lambda i: (i, 0)),
            pl.BlockSpec((_MB, 128), lambda i: (i, 0)),
            pl.BlockSpec((_MK, DM), lambda i: (i, 0)),
            pl.BlockSpec((_MK, DM), lambda i: (i, 0)),
            pl.BlockSpec((_MK, 128), lambda i: (i, 0)),
            pl.BlockSpec((3, DM), full),
            pl.BlockSpec((1, DM), full),
            pl.BlockSpec((DM, DM), full),
            pl.BlockSpec((1, DM), full),
            pl.BlockSpec((DM, DM), full),
            pl.BlockSpec((1, DM), full),
            pl.BlockSpec((DM, DM), full),
            pl.BlockSpec((1, DM), full),
            pl.BlockSpec((DM, DP), full),
            pl.BlockSpec((1, DP), full),
        ],
        out_specs=[
            pl.BlockSpec((_MB, DP), lambda i: (i, 0)),
            pl.BlockSpec((_MB, K, DM), lambda i: (i, 0, 0)),
        ],
        out_shape=[
            jax.ShapeDtypeStruct((BN, DP), jnp.float32),
            jax.ShapeDtypeStruct((BN, K, DM), jnp.float32),
        ],
    )(q, feat, xyzp, kf, vg, xg, d1_w, d1_b, d2_w, d2_b,
      g1_w, g1_b, g2_w, g2_b, fc2_w, fc2_b)


def kernel(xyz, features, fc1_w, fc1_b, fc2_w, fc2_b, d1_w, d1_b, d2_w, d2_b,
           g1_w, g1_b, g2_w, g2_b, wq, wk, wv):
    feat = features.reshape(BN, DP)
    q, xk, xv = _qkv(feat, fc1_w, fc1_b.reshape(1, DM), wq, wk, wv)

    xyzT = jnp.transpose(xyz, (0, 2, 1))
    idx = _knn(xyz, xyzT)

    xyzp = jnp.pad(xyz.reshape(BN, 3), ((0, 0), (0, 125)))
    kf, vg, xg = _sc_gather(idx.reshape(_TOT), xk, xv, xyzp)

    res, attn = _attn(q, feat, xyzp, kf, vg, xg,
                      d1_w, d1_b.reshape(1, DM), d2_w, d2_b.reshape(1, DM),
                      g1_w, g1_b.reshape(1, DM), g2_w, g2_b.reshape(1, DM),
                      fc2_w, fc2_b.reshape(1, DP))
    return res.reshape(B, N, DP), attn.reshape(B, N, K, DM)
